# baseline (speedup 1.0000x reference)
import jax
import jax.numpy as jnp
from jax.experimental import pallas as pl
from jax.experimental.pallas import tpu as pltpu

_T, _D, _E = 32768, 768, 8
_H = _T // 2
_BT = 1024
_NBUF = 4
_NSTEP = _H // _BT


def _select_store(logits, wout_ref, iout_ref):
    lT = jnp.transpose(logits)
    row = jax.lax.broadcasted_iota(jnp.int32, lT.shape, 0)
    m1 = jnp.max(lT, axis=0, keepdims=True)
    i1 = jnp.min(jnp.where(lT == m1, row, _E), axis=0, keepdims=True)
    neg = jnp.full_like(lT, -jnp.inf)
    rest = jnp.where(row == i1, neg, lT)
    m2 = jnp.max(rest, axis=0, keepdims=True)
    i2 = jnp.min(jnp.where(rest == m2, row, _E), axis=0, keepdims=True)
    e = jnp.exp(m2 - m1)
    w1 = 1.0 / (1.0 + e)
    w_pair = jnp.concatenate([w1, 1.0 - w1], axis=0)
    i_pair = jnp.concatenate([i1, i2], axis=0)
    wout_ref[...] = jnp.transpose(w_pair)
    iout_ref[...] = jnp.transpose(i_pair)


def _start(x_hbm, buf, sem, chunk):
    slot = jax.lax.rem(chunk, _NBUF)
    pltpu.make_async_copy(
        x_hbm.at[pl.ds(_H + chunk * _BT, _BT), :], buf.at[slot], sem.at[slot]
    ).start()


def _router_body(xa_ref, xm_hbm, w_ref, wa_ref, ia_ref, wb_ref, ib_ref,
                 buf, sem):
    i = pl.program_id(0)

    @pl.when(i == 0)
    def _prologue():
        for c in range(_NBUF - 1):
            _start(xm_hbm, buf, sem, jnp.int32(c))

    @pl.when(i + (_NBUF - 1) < _NSTEP)
    def _next():
        _start(xm_hbm, buf, sem, i + (_NBUF - 1))

    w = w_ref[...]
    dims = (((1,), (0,)), ((), ()))
    logits_a = jax.lax.dot_general(
        xa_ref[...], w, dims, preferred_element_type=jnp.float32)
    _select_store(logits_a, wa_ref, ia_ref)

    slot = jax.lax.rem(i, _NBUF)
    pltpu.make_async_copy(
        xm_hbm.at[pl.ds(_H + i * _BT, _BT), :], buf.at[slot], sem.at[slot]
    ).wait()
    logits_b = jax.lax.dot_general(
        buf[slot], w, dims, preferred_element_type=jnp.float32)
    _select_store(logits_b, wb_ref, ib_ref)


def kernel(x_TD, kernel_DE):
    x = jnp.asarray(x_TD, jnp.float32)
    w = jnp.asarray(kernel_DE, jnp.float32)
    wa, ia, wb, ib = pl.pallas_call(
        _router_body,
        grid=(_NSTEP,),
        in_specs=[
            pl.BlockSpec((_BT, _D), lambda i: (i, 0)),
            pl.BlockSpec(memory_space=pl.ANY),
            pl.BlockSpec((_D, _E), lambda i: (0, 0)),
        ],
        out_specs=[
            pl.BlockSpec((_BT, 2), lambda i: (i, 0)),
            pl.BlockSpec((_BT, 2), lambda i: (i, 0)),
            pl.BlockSpec((_BT, 2), lambda i: (i, 0)),
            pl.BlockSpec((_BT, 2), lambda i: (i, 0)),
        ],
        out_shape=[
            jax.ShapeDtypeStruct((_H, 2), jnp.float32),
            jax.ShapeDtypeStruct((_H, 2), jnp.int32),
            jax.ShapeDtypeStruct((_H, 2), jnp.float32),
            jax.ShapeDtypeStruct((_H, 2), jnp.int32),
        ],
        scratch_shapes=[
            pltpu.VMEM((_NBUF, _BT, _D), jnp.float32),
            pltpu.SemaphoreType.DMA((_NBUF,)),
        ],
        compiler_params=pltpu.CompilerParams(
            dimension_semantics=("arbitrary",)
        ),
    )(x, x, w)
    weights = jnp.concatenate([wa, wb], axis=0)
    experts = jnp.concatenate([ia, ib], axis=0)
    return (weights, experts)


# speedup vs baseline: 1.1324x; 1.1324x over previous
import jax
import jax.numpy as jnp
from jax.experimental import pallas as pl
from jax.experimental.pallas import tpu as pltpu

_T, _D, _E = 32768, 768, 8
_BT = 1024
_NBUF = 4
_NCHUNK = _T // _BT


def _start(x_hbm, buf, sem, chunk):
    slot = jax.lax.rem(chunk, _NBUF)
    pltpu.make_async_copy(
        x_hbm.at[pl.ds(chunk * _BT, _BT), :], buf.at[slot], sem.at[slot]
    ).start()


def _router_body(x_hbm, w_ref, wout_ref, iout_ref, buf, sem):
    i = pl.program_id(0)

    @pl.when(i == 0)
    def _prologue():
        for c in range(_NBUF - 1):
            _start(x_hbm, buf, sem, jnp.int32(c))

    @pl.when(i + (_NBUF - 1) < _NCHUNK)
    def _next():
        _start(x_hbm, buf, sem, i + (_NBUF - 1))

    slot = jax.lax.rem(i, _NBUF)
    pltpu.make_async_copy(
        x_hbm.at[pl.ds(i * _BT, _BT), :], buf.at[slot], sem.at[slot]
    ).wait()

    x = buf[slot]
    w = w_ref[...]
    logits = jax.lax.dot_general(
        x, w, (((1,), (0,)), ((), ())), preferred_element_type=jnp.float32
    )
    lT = jnp.transpose(logits)
    row = jax.lax.broadcasted_iota(jnp.int32, lT.shape, 0)
    m1 = jnp.max(lT, axis=0, keepdims=True)
    i1 = jnp.min(jnp.where(lT == m1, row, _E), axis=0, keepdims=True)
    neg = jnp.full_like(lT, -jnp.inf)
    rest = jnp.where(row == i1, neg, lT)
    m2 = jnp.max(rest, axis=0, keepdims=True)
    i2 = jnp.min(jnp.where(rest == m2, row, _E), axis=0, keepdims=True)
    e = jnp.exp(m2 - m1)
    w1 = 1.0 / (1.0 + e)
    w_pair = jnp.concatenate([w1, 1.0 - w1], axis=0)
    i_pair = jnp.concatenate([i1, i2], axis=0)
    wout_ref[...] = jnp.transpose(w_pair)
    iout_ref[...] = jnp.transpose(i_pair)


def kernel(x_TD, kernel_DE):
    x = jnp.asarray(x_TD, jnp.float32)
    w = jnp.asarray(kernel_DE, jnp.float32)
    weights, experts = pl.pallas_call(
        _router_body,
        grid=(_NCHUNK,),
        in_specs=[
            pl.BlockSpec(memory_space=pl.ANY),
            pl.BlockSpec((_D, _E), lambda i: (0, 0)),
        ],
        out_specs=[
            pl.BlockSpec((_BT, 2), lambda i: (i, 0)),
            pl.BlockSpec((_BT, 2), lambda i: (i, 0)),
        ],
        out_shape=[
            jax.ShapeDtypeStruct((_T, 2), jnp.float32),
            jax.ShapeDtypeStruct((_T, 2), jnp.int32),
        ],
        scratch_shapes=[
            pltpu.VMEM((_NBUF, _BT, _D), jnp.float32),
            pltpu.SemaphoreType.DMA((_NBUF,)),
        ],
        compiler_params=pltpu.CompilerParams(
            dimension_semantics=("arbitrary",)
        ),
    )(x, w)
    return (weights, experts)
